# Initial kernel scaffold; baseline (speedup 1.0000x reference)
import jax
import jax.numpy as jnp
from jax.experimental import pallas as pl


def kernel(x, edge_index, W_l1, W_r1, b1, W_l2, W_r2, b2, W3, b3):
    raise NotImplementedError("write your pallas kernel here")


# speedup vs baseline: 5.6454x; 5.6454x over previous
import functools

import jax
import jax.numpy as jnp
from jax import lax
from jax.experimental import pallas as pl
from jax.experimental.pallas import tpu as pltpu
from jax.experimental.pallas import tpu_sc as plsc

N_NODES = 10000
N_EDGES = 320000
NC = 2
NS = 16
N_TILES = NC * NS
E_PER_TILE = N_EDGES // N_TILES
CH = 80
N_CH = E_PER_TILE // CH
ROWS_PER_TILE = N_NODES // NS

_BLK = 1000
_GRID = N_NODES // _BLK


def _make_sc_agg(d):
    mesh = plsc.VectorSubcoreMesh(
        core_axis_name="c", subcore_axis_name="s", num_cores=NC, num_subcores=NS)

    @functools.partial(
        pl.kernel,
        out_type=jax.ShapeDtypeStruct((NC, NS, ROWS_PER_TILE, d), jnp.float32),
        mesh=mesh,
        scratch_types=[
            pltpu.VMEM((CH,), jnp.int32),
            pltpu.VMEM((CH,), jnp.int32),
            pltpu.VMEM((CH, d), jnp.float32),
            pltpu.VMEM_SHARED((N_NODES, d), jnp.float32),
            pltpu.SemaphoreType.DMA,
        ],
        compiler_params=pltpu.CompilerParams(use_tc_tiling_on_sc=False),
    )
    def sc_agg(feat, src, dst, zrows, agg_out, srcv, dstv, rowsv, agg_sh, sem):
        c = lax.axis_index("c")
        s = lax.axis_index("s")
        wid = s * NC + c
        r0 = s * ROWS_PER_TILE
        pltpu.sync_copy(zrows, agg_sh.at[pl.ds(r0, ROWS_PER_TILE)])
        plsc.subcore_barrier()

        def body(i, carry):
            base = wid * E_PER_TILE + i * CH
            pltpu.sync_copy(src.at[pl.ds(base, CH)], srcv)
            pltpu.sync_copy(dst.at[pl.ds(base, CH)], dstv)
            pltpu.async_copy(feat.at[srcv], rowsv, sem).wait()
            pltpu.sync_copy(rowsv, agg_sh.at[dstv], add=True)
            return carry

        lax.fori_loop(0, N_CH, body, 0)
        plsc.subcore_barrier()
        pltpu.sync_copy(agg_sh.at[pl.ds(r0, ROWS_PER_TILE)], agg_out.at[c, s])

    return sc_agg


_sc_agg80 = _make_sc_agg(80)
_sc_agg32 = _make_sc_agg(32)


def _tc1_body(x_ref, wa_ref, wr_ref, p_ref, r_ref):
    xb = x_ref[...]
    p = jnp.dot(xb, wa_ref[...], preferred_element_type=jnp.float32)
    ones_col = (lax.broadcasted_iota(jnp.int32, p.shape, 1) == 64).astype(jnp.float32)
    p_ref[...] = p + ones_col
    r_ref[...] = jnp.dot(xb, wr_ref[...], preferred_element_type=jnp.float32)


def _tc1(x, wl1aug, wr1):
    return pl.pallas_call(
        _tc1_body,
        grid=(_GRID,),
        in_specs=[
            pl.BlockSpec((_BLK, 128), lambda i: (i, 0)),
            pl.BlockSpec((128, 80), lambda i: (0, 0)),
            pl.BlockSpec((128, 64), lambda i: (0, 0)),
        ],
        out_specs=[
            pl.BlockSpec((_BLK, 80), lambda i: (i, 0)),
            pl.BlockSpec((_BLK, 64), lambda i: (i, 0)),
        ],
        out_shape=[
            jax.ShapeDtypeStruct((N_NODES, 80), jnp.float32),
            jax.ShapeDtypeStruct((N_NODES, 64), jnp.float32),
        ],
    )(x, wl1aug, wr1)


def _tc2_body(agg_ref, r1_ref, b1_ref, wl2_ref, wr2_ref,
              p2_ref, r2_ref, inv8_ref):
    aggs = agg_ref[0] + agg_ref[1]
    deg = jnp.sum(aggs[:, 64:], axis=-1, keepdims=True)
    inv = 1.0 / jnp.maximum(deg, 1.0)
    mean = aggs[:, :64] * inv
    h = jnp.maximum(mean + r1_ref[...] + b1_ref[...], 0.0)
    p2_ref[...] = jnp.dot(h, wl2_ref[...], preferred_element_type=jnp.float32)
    r2_ref[...] = jnp.dot(h, wr2_ref[...], preferred_element_type=jnp.float32)
    inv8_ref[...] = jnp.broadcast_to(inv, (inv.shape[0], 8))


def _tc2(agg1, r1, b1r, wl2, wr2):
    return pl.pallas_call(
        _tc2_body,
        grid=(_GRID,),
        in_specs=[
            pl.BlockSpec((NC, _BLK, 80), lambda i: (0, i, 0)),
            pl.BlockSpec((_BLK, 64), lambda i: (i, 0)),
            pl.BlockSpec((1, 64), lambda i: (0, 0)),
            pl.BlockSpec((64, 32), lambda i: (0, 0)),
            pl.BlockSpec((64, 32), lambda i: (0, 0)),
        ],
        out_specs=[
            pl.BlockSpec((_BLK, 32), lambda i: (i, 0)),
            pl.BlockSpec((_BLK, 32), lambda i: (i, 0)),
            pl.BlockSpec((_BLK, 8), lambda i: (i, 0)),
        ],
        out_shape=[
            jax.ShapeDtypeStruct((N_NODES, 32), jnp.float32),
            jax.ShapeDtypeStruct((N_NODES, 32), jnp.float32),
            jax.ShapeDtypeStruct((N_NODES, 8), jnp.float32),
        ],
    )(agg1, r1, b1r, wl2, wr2)


def _tc3_body(agg_ref, r2_ref, inv8_ref, b2_ref, w3_ref, b3_ref, o_ref):
    inv = inv8_ref[...][:, :1]
    h2 = jnp.maximum((agg_ref[0] + agg_ref[1]) * inv + r2_ref[...] + b2_ref[...], 0.0)
    o_ref[...] = jnp.dot(h2, w3_ref[...], preferred_element_type=jnp.float32) + b3_ref[...]


def _tc3(agg2, r2, inv8, b2r, w3pad, b3r):
    return pl.pallas_call(
        _tc3_body,
        grid=(_GRID,),
        in_specs=[
            pl.BlockSpec((NC, _BLK, 32), lambda i: (0, i, 0)),
            pl.BlockSpec((_BLK, 32), lambda i: (i, 0)),
            pl.BlockSpec((_BLK, 8), lambda i: (i, 0)),
            pl.BlockSpec((1, 32), lambda i: (0, 0)),
            pl.BlockSpec((32, 128), lambda i: (0, 0)),
            pl.BlockSpec((1, 128), lambda i: (0, 0)),
        ],
        out_specs=pl.BlockSpec((_BLK, 128), lambda i: (i, 0)),
        out_shape=jax.ShapeDtypeStruct((N_NODES, 128), jnp.float32),
    )(agg2, r2, inv8, b2r, w3pad, b3r)


def kernel(x, edge_index, W_l1, W_r1, b1, W_l2, W_r2, b2, W3, b3):
    src = edge_index[0].astype(jnp.int32)
    dst = edge_index[1].astype(jnp.int32)
    wl1aug = jnp.pad(W_l1, ((0, 0), (0, 16)))
    z80 = jnp.zeros((ROWS_PER_TILE, 80), jnp.float32)
    z32 = jnp.zeros((ROWS_PER_TILE, 32), jnp.float32)

    p1aug, r1 = _tc1(x, wl1aug, W_r1)
    agg1 = _sc_agg80(p1aug, src, dst, z80).reshape(NC, N_NODES, 80)
    p2, r2, inv8 = _tc2(agg1, r1, b1.reshape(1, 64), W_l2, W_r2)
    agg2 = _sc_agg32(p2, src, dst, z32).reshape(NC, N_NODES, 32)
    o = _tc3(agg2, r2, inv8, b2.reshape(1, 32),
             jnp.pad(W3, ((0, 0), (0, 127))),
             jnp.broadcast_to(b3.reshape(1, 1), (1, 128)))
    return o[:, :1]


# speedup vs baseline: 6.8378x; 1.2112x over previous
import functools

import jax
import jax.numpy as jnp
from jax import lax
from jax.experimental import pallas as pl
from jax.experimental.pallas import tpu as pltpu
from jax.experimental.pallas import tpu_sc as plsc

N_NODES = 10000
N_EDGES = 320000
NC = 2
NS = 16
N_TILES = NC * NS
E_PER_TILE = N_EDGES // N_TILES
CH = 128
N_CH = 80
E_PAD_TILE = N_CH * CH
ROWS_PER_TILE = N_NODES // NS
ACC_ROWS = N_NODES + 8

_BLK = 1000
_GRID = N_NODES // _BLK


def _make_sc_agg(d):
    mesh = plsc.VectorSubcoreMesh(
        core_axis_name="c", subcore_axis_name="s", num_cores=NC, num_subcores=NS)

    @functools.partial(
        pl.kernel,
        out_type=jax.ShapeDtypeStruct((NC, NS, ROWS_PER_TILE, d), jnp.float32),
        mesh=mesh,
        scratch_types=[
            pltpu.VMEM((N_CH, CH), jnp.int32),
            pltpu.VMEM((N_CH, CH), jnp.int32),
            pltpu.VMEM((CH, d), jnp.float32),
            pltpu.VMEM((CH, d), jnp.float32),
            pltpu.VMEM_SHARED((ACC_ROWS, d), jnp.float32),
            pltpu.SemaphoreType.DMA,
            pltpu.SemaphoreType.DMA,
        ],
        compiler_params=pltpu.CompilerParams(use_tc_tiling_on_sc=False),
    )
    def sc_agg(feat, src, dst, zrows, agg_out, srcv, dstv, rows0, rows1, agg_sh,
               sem0, sem1):
        c = lax.axis_index("c")
        s = lax.axis_index("s")
        wid = s * NC + c
        r0 = s * ROWS_PER_TILE
        pltpu.sync_copy(zrows, agg_sh.at[pl.ds(r0, ROWS_PER_TILE)])
        pltpu.sync_copy(src.at[wid], srcv)
        pltpu.sync_copy(dst.at[wid], dstv)
        plsc.subcore_barrier()

        pltpu.async_copy(feat.at[srcv.at[0]], rows0, sem0)

        def drain(buf, sem):
            pltpu.make_async_copy(feat.at[pl.ds(0, CH)], buf, sem).wait()

        def body(j, carry):
            k = 2 * j
            pltpu.async_copy(feat.at[srcv.at[k + 1]], rows1, sem1)
            drain(rows0, sem0)
            pltpu.sync_copy(rows0, agg_sh.at[dstv.at[k]], add=True)

            @pl.when(j < N_CH // 2 - 1)
            def _():
                pltpu.async_copy(feat.at[srcv.at[k + 2]], rows0, sem0)

            drain(rows1, sem1)
            pltpu.sync_copy(rows1, agg_sh.at[dstv.at[k + 1]], add=True)
            return carry

        lax.fori_loop(0, N_CH // 2, body, 0)
        plsc.subcore_barrier()
        pltpu.sync_copy(agg_sh.at[pl.ds(r0, ROWS_PER_TILE)], agg_out.at[c, s])

    return sc_agg


_sc_agg80 = _make_sc_agg(80)
_sc_agg32 = _make_sc_agg(32)


def _tc1_body(x_ref, wa_ref, wr_ref, p_ref, r_ref):
    xb = x_ref[...]
    p = jnp.dot(xb, wa_ref[...], preferred_element_type=jnp.float32)
    ones_col = (lax.broadcasted_iota(jnp.int32, p.shape, 1) == 64).astype(jnp.float32)
    p_ref[...] = p + ones_col
    r_ref[...] = jnp.dot(xb, wr_ref[...], preferred_element_type=jnp.float32)


def _tc1(x, wl1aug, wr1):
    return pl.pallas_call(
        _tc1_body,
        grid=(_GRID,),
        in_specs=[
            pl.BlockSpec((_BLK, 128), lambda i: (i, 0)),
            pl.BlockSpec((128, 80), lambda i: (0, 0)),
            pl.BlockSpec((128, 64), lambda i: (0, 0)),
        ],
        out_specs=[
            pl.BlockSpec((_BLK, 80), lambda i: (i, 0)),
            pl.BlockSpec((_BLK, 64), lambda i: (i, 0)),
        ],
        out_shape=[
            jax.ShapeDtypeStruct((N_NODES, 80), jnp.float32),
            jax.ShapeDtypeStruct((N_NODES, 64), jnp.float32),
        ],
    )(x, wl1aug, wr1)


def _tc2_body(agg_ref, r1_ref, b1_ref, wl2_ref, wr2_ref,
              p2_ref, r2_ref, inv8_ref):
    aggs = agg_ref[0] + agg_ref[1]
    deg = jnp.sum(aggs[:, 64:], axis=-1, keepdims=True)
    inv = 1.0 / jnp.maximum(deg, 1.0)
    mean = aggs[:, :64] * inv
    h = jnp.maximum(mean + r1_ref[...] + b1_ref[...], 0.0)
    p2_ref[...] = jnp.dot(h, wl2_ref[...], preferred_element_type=jnp.float32)
    r2_ref[...] = jnp.dot(h, wr2_ref[...], preferred_element_type=jnp.float32)
    inv8_ref[...] = jnp.broadcast_to(inv, (inv.shape[0], 8))


def _tc2(agg1, r1, b1r, wl2, wr2):
    return pl.pallas_call(
        _tc2_body,
        grid=(_GRID,),
        in_specs=[
            pl.BlockSpec((NC, _BLK, 80), lambda i: (0, i, 0)),
            pl.BlockSpec((_BLK, 64), lambda i: (i, 0)),
            pl.BlockSpec((1, 64), lambda i: (0, 0)),
            pl.BlockSpec((64, 32), lambda i: (0, 0)),
            pl.BlockSpec((64, 32), lambda i: (0, 0)),
        ],
        out_specs=[
            pl.BlockSpec((_BLK, 32), lambda i: (i, 0)),
            pl.BlockSpec((_BLK, 32), lambda i: (i, 0)),
            pl.BlockSpec((_BLK, 8), lambda i: (i, 0)),
        ],
        out_shape=[
            jax.ShapeDtypeStruct((N_NODES, 32), jnp.float32),
            jax.ShapeDtypeStruct((N_NODES, 32), jnp.float32),
            jax.ShapeDtypeStruct((N_NODES, 8), jnp.float32),
        ],
    )(agg1, r1, b1r, wl2, wr2)


def _tc3_body(agg_ref, r2_ref, inv8_ref, b2_ref, w3_ref, b3_ref, o_ref):
    inv = inv8_ref[...][:, :1]
    h2 = jnp.maximum((agg_ref[0] + agg_ref[1]) * inv + r2_ref[...] + b2_ref[...], 0.0)
    o_ref[...] = jnp.dot(h2, w3_ref[...], preferred_element_type=jnp.float32) + b3_ref[...]


def _tc3(agg2, r2, inv8, b2r, w3pad, b3r):
    return pl.pallas_call(
        _tc3_body,
        grid=(_GRID,),
        in_specs=[
            pl.BlockSpec((NC, _BLK, 32), lambda i: (0, i, 0)),
            pl.BlockSpec((_BLK, 32), lambda i: (i, 0)),
            pl.BlockSpec((_BLK, 8), lambda i: (i, 0)),
            pl.BlockSpec((1, 32), lambda i: (0, 0)),
            pl.BlockSpec((32, 128), lambda i: (0, 0)),
            pl.BlockSpec((1, 128), lambda i: (0, 0)),
        ],
        out_specs=pl.BlockSpec((_BLK, 128), lambda i: (i, 0)),
        out_shape=jax.ShapeDtypeStruct((N_NODES, 128), jnp.float32),
    )(agg2, r2, inv8, b2r, w3pad, b3r)


def kernel(x, edge_index, W_l1, W_r1, b1, W_l2, W_r2, b2, W3, b3):
    pad = E_PAD_TILE - E_PER_TILE
    src = jnp.pad(edge_index[0].astype(jnp.int32).reshape(N_TILES, E_PER_TILE),
                  ((0, 0), (0, pad))).reshape(N_TILES, N_CH, CH)
    dst = jnp.pad(edge_index[1].astype(jnp.int32).reshape(N_TILES, E_PER_TILE),
                  ((0, 0), (0, pad)),
                  constant_values=N_NODES).reshape(N_TILES, N_CH, CH)
    wl1aug = jnp.pad(W_l1, ((0, 0), (0, 16)))
    z80 = jnp.zeros((ROWS_PER_TILE, 80), jnp.float32)
    z32 = jnp.zeros((ROWS_PER_TILE, 32), jnp.float32)

    p1aug, r1 = _tc1(x, wl1aug, W_r1)
    agg1 = _sc_agg80(p1aug, src, dst, z80).reshape(NC, N_NODES, 80)
    p2, r2, inv8 = _tc2(agg1, r1, b1.reshape(1, 64), W_l2, W_r2)
    agg2 = _sc_agg32(p2, src, dst, z32).reshape(NC, N_NODES, 32)
    o = _tc3(agg2, r2, inv8, b2.reshape(1, 32),
             jnp.pad(W3, ((0, 0), (0, 127))),
             jnp.broadcast_to(b3.reshape(1, 1), (1, 128)))
    return o[:, :1]


# speedup vs baseline: 8.0687x; 1.1800x over previous
import functools

import jax
import jax.numpy as jnp
from jax import lax
from jax.experimental import pallas as pl
from jax.experimental.pallas import tpu as pltpu
from jax.experimental.pallas import tpu_sc as plsc

N_NODES = 10000
N_EDGES = 320000
NC = 2
NS = 16
N_TILES = NC * NS
E_PER_TILE = N_EDGES // N_TILES
CH = 128
N_CH = 80
E_PAD_TILE = N_CH * CH
ROWS_PER_TILE = N_NODES // NS
ACC_ROWS = N_NODES + 8

_BLK = 1000
_GRID = N_NODES // _BLK


def _make_sc_agg(d, spmem_table):
    mesh = plsc.VectorSubcoreMesh(
        core_axis_name="c", subcore_axis_name="s", num_cores=NC, num_subcores=NS)

    scratch = [
        pltpu.VMEM((N_CH, CH), jnp.int32),
        pltpu.VMEM((N_CH, CH), jnp.int32),
        pltpu.VMEM((N_CH, CH), jnp.int32),
        pltpu.VMEM((CH, d), jnp.float32),
        pltpu.VMEM((CH, d), jnp.float32),
        pltpu.VMEM_SHARED((ACC_ROWS, d), jnp.float32),
        pltpu.SemaphoreType.DMA,
        pltpu.SemaphoreType.DMA,
    ]
    if spmem_table:
        scratch.insert(6, pltpu.VMEM_SHARED((N_NODES, d), jnp.float32))

    @functools.partial(
        pl.kernel,
        out_type=jax.ShapeDtypeStruct((NC, NS, ROWS_PER_TILE, d), jnp.float32),
        mesh=mesh,
        scratch_types=scratch,
        compiler_params=pltpu.CompilerParams(use_tc_tiling_on_sc=False),
    )
    def sc_agg(feat, edges, zrows, agg_out, epk, srcv, dstv, rows0, rows1, agg_sh,
               *rest):
        if spmem_table:
            feat_sh, sem0, sem1 = rest
        else:
            sem0, sem1 = rest
            feat_sh = feat
        c = lax.axis_index("c")
        s = lax.axis_index("s")
        wid = s * NC + c
        r0 = s * ROWS_PER_TILE
        pltpu.sync_copy(zrows, agg_sh.at[pl.ds(r0, ROWS_PER_TILE)])
        if spmem_table:
            pltpu.sync_copy(feat.at[pl.ds(r0, ROWS_PER_TILE)],
                            feat_sh.at[pl.ds(r0, ROWS_PER_TILE)])
        pltpu.sync_copy(edges.at[wid], epk)

        def unpack(g, carry):
            row = g >> 3
            col = (g & 7) * 16
            e = epk[row, pl.ds(col, 16)]
            srcv[row, pl.ds(col, 16)] = e & 0xFFFF
            dstv[row, pl.ds(col, 16)] = lax.shift_right_logical(e, 16)
            return carry

        lax.fori_loop(0, N_CH * CH // 16, unpack, 0)
        plsc.subcore_barrier()

        pltpu.async_copy(feat_sh.at[srcv.at[0]], rows0, sem0)

        def drain(buf, sem):
            pltpu.make_async_copy(feat.at[pl.ds(0, CH)], buf, sem).wait()

        def body(j, carry):
            k = 2 * j
            pltpu.async_copy(feat_sh.at[srcv.at[k + 1]], rows1, sem1)
            drain(rows0, sem0)
            pltpu.sync_copy(rows0, agg_sh.at[dstv.at[k]], add=True)

            @pl.when(j < N_CH // 2 - 1)
            def _():
                pltpu.async_copy(feat_sh.at[srcv.at[k + 2]], rows0, sem0)

            drain(rows1, sem1)
            pltpu.sync_copy(rows1, agg_sh.at[dstv.at[k + 1]], add=True)
            return carry

        lax.fori_loop(0, N_CH // 2, body, 0)
        plsc.subcore_barrier()
        pltpu.sync_copy(agg_sh.at[pl.ds(r0, ROWS_PER_TILE)], agg_out.at[c, s])

    return sc_agg


_sc_agg80 = _make_sc_agg(80, spmem_table=False)
_sc_agg32 = _make_sc_agg(32, spmem_table=True)


def _tc1_body(x_ref, wa_ref, wr_ref, p_ref, r_ref):
    xb = x_ref[...]
    p = jnp.dot(xb, wa_ref[...], preferred_element_type=jnp.float32)
    ones_col = (lax.broadcasted_iota(jnp.int32, p.shape, 1) == 64).astype(jnp.float32)
    p_ref[...] = p + ones_col
    r_ref[...] = jnp.dot(xb, wr_ref[...], preferred_element_type=jnp.float32)


def _tc1(x, wl1aug, wr1):
    return pl.pallas_call(
        _tc1_body,
        grid=(_GRID,),
        in_specs=[
            pl.BlockSpec((_BLK, 128), lambda i: (i, 0)),
            pl.BlockSpec((128, 80), lambda i: (0, 0)),
            pl.BlockSpec((128, 64), lambda i: (0, 0)),
        ],
        out_specs=[
            pl.BlockSpec((_BLK, 80), lambda i: (i, 0)),
            pl.BlockSpec((_BLK, 64), lambda i: (i, 0)),
        ],
        out_shape=[
            jax.ShapeDtypeStruct((N_NODES, 80), jnp.float32),
            jax.ShapeDtypeStruct((N_NODES, 64), jnp.float32),
        ],
    )(x, wl1aug, wr1)


def _tc2_body(agg_ref, r1_ref, b1_ref, wl2_ref, wr2_ref,
              p2_ref, r2_ref, inv8_ref):
    aggs = agg_ref[0] + agg_ref[1]
    deg = jnp.sum(aggs[:, 64:], axis=-1, keepdims=True)
    inv = 1.0 / jnp.maximum(deg, 1.0)
    mean = aggs[:, :64] * inv
    h = jnp.maximum(mean + r1_ref[...] + b1_ref[...], 0.0)
    p2_ref[...] = jnp.dot(h, wl2_ref[...], preferred_element_type=jnp.float32)
    r2_ref[...] = jnp.dot(h, wr2_ref[...], preferred_element_type=jnp.float32)
    inv8_ref[...] = jnp.broadcast_to(inv, (inv.shape[0], 8))


def _tc2(agg1, r1, b1r, wl2, wr2):
    return pl.pallas_call(
        _tc2_body,
        grid=(_GRID,),
        in_specs=[
            pl.BlockSpec((NC, _BLK, 80), lambda i: (0, i, 0)),
            pl.BlockSpec((_BLK, 64), lambda i: (i, 0)),
            pl.BlockSpec((1, 64), lambda i: (0, 0)),
            pl.BlockSpec((64, 32), lambda i: (0, 0)),
            pl.BlockSpec((64, 32), lambda i: (0, 0)),
        ],
        out_specs=[
            pl.BlockSpec((_BLK, 32), lambda i: (i, 0)),
            pl.BlockSpec((_BLK, 32), lambda i: (i, 0)),
            pl.BlockSpec((_BLK, 8), lambda i: (i, 0)),
        ],
        out_shape=[
            jax.ShapeDtypeStruct((N_NODES, 32), jnp.float32),
            jax.ShapeDtypeStruct((N_NODES, 32), jnp.float32),
            jax.ShapeDtypeStruct((N_NODES, 8), jnp.float32),
        ],
    )(agg1, r1, b1r, wl2, wr2)


def _tc3_body(agg_ref, r2_ref, inv8_ref, b2_ref, w3_ref, b3_ref, o_ref):
    inv = inv8_ref[...][:, :1]
    h2 = jnp.maximum((agg_ref[0] + agg_ref[1]) * inv + r2_ref[...] + b2_ref[...], 0.0)
    o_ref[...] = jnp.dot(h2, w3_ref[...], preferred_element_type=jnp.float32) + b3_ref[...]


def _tc3(agg2, r2, inv8, b2r, w3pad, b3r):
    return pl.pallas_call(
        _tc3_body,
        grid=(_GRID,),
        in_specs=[
            pl.BlockSpec((NC, _BLK, 32), lambda i: (0, i, 0)),
            pl.BlockSpec((_BLK, 32), lambda i: (i, 0)),
            pl.BlockSpec((_BLK, 8), lambda i: (i, 0)),
            pl.BlockSpec((1, 32), lambda i: (0, 0)),
            pl.BlockSpec((32, 128), lambda i: (0, 0)),
            pl.BlockSpec((1, 128), lambda i: (0, 0)),
        ],
        out_specs=pl.BlockSpec((_BLK, 128), lambda i: (i, 0)),
        out_shape=jax.ShapeDtypeStruct((N_NODES, 128), jnp.float32),
    )(agg2, r2, inv8, b2r, w3pad, b3r)


def kernel(x, edge_index, W_l1, W_r1, b1, W_l2, W_r2, b2, W3, b3):
    pad = E_PAD_TILE - E_PER_TILE
    src = jnp.pad(edge_index[0].astype(jnp.int32).reshape(N_TILES, E_PER_TILE),
                  ((0, 0), (0, pad)))
    dst = jnp.pad(edge_index[1].astype(jnp.int32).reshape(N_TILES, E_PER_TILE),
                  ((0, 0), (0, pad)), constant_values=N_NODES)
    edges = (src | (dst << 16)).reshape(N_TILES, N_CH, CH)
    wl1aug = jnp.pad(W_l1, ((0, 0), (0, 16)))
    z80 = jnp.zeros((ROWS_PER_TILE, 80), jnp.float32)
    z32 = jnp.zeros((ROWS_PER_TILE, 32), jnp.float32)

    p1aug, r1 = _tc1(x, wl1aug, W_r1)
    agg1 = _sc_agg80(p1aug, edges, z80).reshape(NC, N_NODES, 80)
    p2, r2, inv8 = _tc2(agg1, r1, b1.reshape(1, 64), W_l2, W_r2)
    agg2 = _sc_agg32(p2, edges, z32).reshape(NC, N_NODES, 32)
    o = _tc3(agg2, r2, inv8, b2.reshape(1, 32),
             jnp.pad(W3, ((0, 0), (0, 127))),
             jnp.broadcast_to(b3.reshape(1, 1), (1, 128)))
    return o[:, :1]


# speedup vs baseline: 8.8720x; 1.0996x over previous
import functools

import jax
import jax.numpy as jnp
from jax import lax
from jax.experimental import pallas as pl
from jax.experimental.pallas import tpu as pltpu
from jax.experimental.pallas import tpu_sc as plsc

N_NODES = 10000
N_EDGES = 320000
NC = 2
NS = 16
N_TILES = NC * NS
E_PER_TILE = N_EDGES // N_TILES
CH = 128
N_CH = 80
E_PAD_TILE = N_CH * CH
ROWS_PER_TILE = N_NODES // NS
ACC_ROWS = N_NODES + 8
DEG_ROWS = 640

_BLK = 1000
_GRID = N_NODES // _BLK


def _make_sc_agg(d, compute_deg):
    mesh = plsc.VectorSubcoreMesh(
        core_axis_name="c", subcore_axis_name="s", num_cores=NC, num_subcores=NS)

    out_type = [jax.ShapeDtypeStruct((NC, NS, ROWS_PER_TILE, d), jnp.float32)]
    scratch = [
        pltpu.VMEM((N_CH, CH), jnp.int32),
        pltpu.VMEM((N_CH, CH), jnp.int32),
        pltpu.VMEM((N_CH, CH), jnp.int32),
        pltpu.VMEM((CH, d), jnp.float32),
        pltpu.VMEM((CH, d), jnp.float32),
        pltpu.VMEM_SHARED((ACC_ROWS, d), jnp.float32),
        pltpu.SemaphoreType.DMA,
        pltpu.SemaphoreType.DMA,
    ]
    if not compute_deg:
        scratch.insert(5, pltpu.VMEM_SHARED((N_NODES, d), jnp.float32))
    if compute_deg:
        out_type.append(jax.ShapeDtypeStruct((NC, DEG_ROWS, 16), jnp.float32))
        scratch += [
            pltpu.VMEM((DEG_ROWS, 16), jnp.float32),
            pltpu.VMEM((DEG_ROWS // CH, CH), jnp.int32),
            pltpu.VMEM_SHARED((DEG_ROWS, 16), jnp.float32),
        ]

    @functools.partial(
        pl.kernel,
        out_type=out_type,
        mesh=mesh,
        scratch_types=scratch,
        compiler_params=pltpu.CompilerParams(use_tc_tiling_on_sc=False,
                                            needs_layout_passes=False),
    )
    def sc_agg(feat, edges, *refs):
        if compute_deg:
            (agg_out, deg_out, epk, srcv, dstv, rows0, rows1, agg_sh,
             sem0, sem1, deg2d, rowidx, deg_sh) = refs
            feat_sh = feat
        else:
            (agg_out, epk, srcv, dstv, rows0, rows1, feat_sh, agg_sh,
             sem0, sem1) = refs
        c = lax.axis_index("c")
        s = lax.axis_index("s")
        wid = s * NC + c
        r0 = s * ROWS_PER_TILE
        z16 = jnp.zeros((16,), jnp.float32)

        if feat_sh is not feat:
            pltpu.sync_copy(feat.at[pl.ds(r0, ROWS_PER_TILE)],
                            feat_sh.at[pl.ds(r0, ROWS_PER_TILE)])
        pltpu.sync_copy(edges.at[wid], epk)

        n16 = d // 16

        def zrow(g, carry):
            rows0[lax.div(g, n16), pl.ds(lax.rem(g, n16) * 16, 16)] = z16
            return carry

        lax.fori_loop(0, CH * n16, zrow, 0)
        off = 0
        while off < ROWS_PER_TILE:
            sz = min(CH, ROWS_PER_TILE - off)
            pltpu.sync_copy(rows0.at[pl.ds(0, sz)],
                            agg_sh.at[pl.ds(r0 + off, sz)])
            off += sz

        if compute_deg:
            def zdeg(i, carry):
                deg2d[i] = z16
                return carry

            lax.fori_loop(0, DEG_ROWS, zdeg, 0)

            @pl.when(s == 0)
            def _():
                pltpu.sync_copy(deg2d, deg_sh)

            def irow(g, carry):
                rowidx[lax.div(g, 8), pl.ds(lax.rem(g, 8) * 16, 16)] = (
                    lax.iota(jnp.int32, 16) + g * 16)
                return carry

            lax.fori_loop(0, DEG_ROWS // 16, irow, 0)

        def unpack(g, carry):
            row = lax.div(g, 8)
            col = lax.rem(g, 8) * 16
            e = epk[row, pl.ds(col, 16)]
            srcv[row, pl.ds(col, 16)] = e & 0xFFFF
            dstv[row, pl.ds(col, 16)] = lax.shift_right_logical(e, 16)
            return carry

        lax.fori_loop(0, N_CH * CH // 16, unpack, 0)
        plsc.subcore_barrier()

        pltpu.async_copy(feat_sh.at[srcv.at[0]], rows0, sem0)

        def drain(buf, sem):
            pltpu.make_async_copy(feat.at[pl.ds(0, CH)], buf, sem).wait()

        def hist(k):
            for g in range(CH // 16):
                di = dstv[k, pl.ds(g * 16, 16)]
                q = lax.shift_right_logical(di, 4)
                r = di & 15
                plsc.addupdate_scatter(deg2d, [q, r], jnp.ones((16,), jnp.float32))

        def body(j, carry):
            k = 2 * j
            pltpu.async_copy(feat_sh.at[srcv.at[k + 1]], rows1, sem1)
            if compute_deg:
                hist(k)
            drain(rows0, sem0)
            pltpu.sync_copy(rows0, agg_sh.at[dstv.at[k]], add=True)

            @pl.when(j < N_CH // 2 - 1)
            def _():
                pltpu.async_copy(feat_sh.at[srcv.at[k + 2]], rows0, sem0)

            if compute_deg:
                hist(k + 1)
            drain(rows1, sem1)
            pltpu.sync_copy(rows1, agg_sh.at[dstv.at[k + 1]], add=True)
            return carry

        lax.fori_loop(0, N_CH // 2, body, 0)

        if compute_deg:
            for t in range(DEG_ROWS // CH):
                pltpu.sync_copy(deg2d.at[pl.ds(t * CH, CH)],
                                deg_sh.at[rowidx.at[t]], add=True)
        plsc.subcore_barrier()
        pltpu.sync_copy(agg_sh.at[pl.ds(r0, ROWS_PER_TILE)], agg_out.at[c, s])
        if compute_deg:
            @pl.when(s == 0)
            def _():
                pltpu.sync_copy(deg_sh, deg_out.at[c])

    return sc_agg


_sc_agg64 = _make_sc_agg(64, compute_deg=True)
_sc_agg32 = _make_sc_agg(32, compute_deg=False)


def _tc1_body(x_ref, wl_ref, wr_ref, p_ref, r_ref):
    xb = x_ref[...]
    p_ref[...] = jnp.dot(xb, wl_ref[...], preferred_element_type=jnp.float32)
    r_ref[...] = jnp.dot(xb, wr_ref[...], preferred_element_type=jnp.float32)


def _tc1(x, wl1, wr1):
    return pl.pallas_call(
        _tc1_body,
        grid=(_GRID,),
        in_specs=[
            pl.BlockSpec((_BLK, 128), lambda i: (i, 0)),
            pl.BlockSpec((128, 64), lambda i: (0, 0)),
            pl.BlockSpec((128, 64), lambda i: (0, 0)),
        ],
        out_specs=[
            pl.BlockSpec((_BLK, 64), lambda i: (i, 0)),
            pl.BlockSpec((_BLK, 64), lambda i: (i, 0)),
        ],
        out_shape=[
            jax.ShapeDtypeStruct((N_NODES, 64), jnp.float32),
            jax.ShapeDtypeStruct((N_NODES, 64), jnp.float32),
        ],
    )(x, wl1, wr1)


def _tc2_body(agg_ref, deg_ref, r1_ref, b1_ref, wl2_ref, wr2_ref,
              p2_ref, r2_ref, inv8_ref):
    deg = deg_ref[0] + deg_ref[1]
    inv = 1.0 / jnp.maximum(deg, 1.0)
    mean = (agg_ref[0] + agg_ref[1]) * inv
    h = jnp.maximum(mean + r1_ref[...] + b1_ref[...], 0.0)
    p2_ref[...] = jnp.dot(h, wl2_ref[...], preferred_element_type=jnp.float32)
    r2_ref[...] = jnp.dot(h, wr2_ref[...], preferred_element_type=jnp.float32)
    inv8_ref[...] = jnp.broadcast_to(inv, (inv.shape[0], 8))


def _tc2(agg1, deg3, r1, b1r, wl2, wr2):
    return pl.pallas_call(
        _tc2_body,
        grid=(_GRID,),
        in_specs=[
            pl.BlockSpec((NC, _BLK, 64), lambda i: (0, i, 0)),
            pl.BlockSpec((NC, _BLK, 1), lambda i: (0, i, 0)),
            pl.BlockSpec((_BLK, 64), lambda i: (i, 0)),
            pl.BlockSpec((1, 64), lambda i: (0, 0)),
            pl.BlockSpec((64, 32), lambda i: (0, 0)),
            pl.BlockSpec((64, 32), lambda i: (0, 0)),
        ],
        out_specs=[
            pl.BlockSpec((_BLK, 32), lambda i: (i, 0)),
            pl.BlockSpec((_BLK, 32), lambda i: (i, 0)),
            pl.BlockSpec((_BLK, 8), lambda i: (i, 0)),
        ],
        out_shape=[
            jax.ShapeDtypeStruct((N_NODES, 32), jnp.float32),
            jax.ShapeDtypeStruct((N_NODES, 32), jnp.float32),
            jax.ShapeDtypeStruct((N_NODES, 8), jnp.float32),
        ],
    )(agg1, deg3, r1, b1r, wl2, wr2)


def _tc3_body(agg_ref, r2_ref, inv8_ref, b2_ref, w3_ref, b3_ref, o_ref):
    inv = inv8_ref[...][:, :1]
    h2 = jnp.maximum((agg_ref[0] + agg_ref[1]) * inv + r2_ref[...] + b2_ref[...], 0.0)
    o_ref[...] = jnp.dot(h2, w3_ref[...], preferred_element_type=jnp.float32) + b3_ref[...]


def _tc3(agg2, r2, inv8, b2r, w3pad, b3r):
    return pl.pallas_call(
        _tc3_body,
        grid=(_GRID,),
        in_specs=[
            pl.BlockSpec((NC, _BLK, 32), lambda i: (0, i, 0)),
            pl.BlockSpec((_BLK, 32), lambda i: (i, 0)),
            pl.BlockSpec((_BLK, 8), lambda i: (i, 0)),
            pl.BlockSpec((1, 32), lambda i: (0, 0)),
            pl.BlockSpec((32, 128), lambda i: (0, 0)),
            pl.BlockSpec((1, 128), lambda i: (0, 0)),
        ],
        out_specs=pl.BlockSpec((_BLK, 128), lambda i: (i, 0)),
        out_shape=jax.ShapeDtypeStruct((N_NODES, 128), jnp.float32),
    )(agg2, r2, inv8, b2r, w3pad, b3r)


def kernel(x, edge_index, W_l1, W_r1, b1, W_l2, W_r2, b2, W3, b3):
    pad = E_PAD_TILE - E_PER_TILE
    src = jnp.pad(edge_index[0].astype(jnp.int32).reshape(N_TILES, E_PER_TILE),
                  ((0, 0), (0, pad)))
    dst = jnp.pad(edge_index[1].astype(jnp.int32).reshape(N_TILES, E_PER_TILE),
                  ((0, 0), (0, pad)), constant_values=N_NODES)
    edges = (src | (dst << 16)).reshape(N_TILES, N_CH, CH)

    p1, r1 = _tc1(x, W_l1, W_r1)
    agg1, deg_raw = _sc_agg64(p1, edges)
    agg1 = agg1.reshape(NC, N_NODES, 64)
    deg3 = deg_raw.reshape(NC, DEG_ROWS * 16, 1)[:, :N_NODES]
    p2, r2, inv8 = _tc2(agg1, deg3, r1, b1.reshape(1, 64), W_l2, W_r2)
    agg2 = _sc_agg32(p2, edges)[0].reshape(NC, N_NODES, 32)
    o = _tc3(agg2, r2, inv8, b2.reshape(1, 32),
             jnp.pad(W3, ((0, 0), (0, 127))),
             jnp.broadcast_to(b3.reshape(1, 1), (1, 128)))
    return o[:, :1]


# speedup vs baseline: 14.0446x; 1.5830x over previous
import functools

import jax
import jax.numpy as jnp
from jax import lax
from jax.experimental import pallas as pl
from jax.experimental.pallas import tpu as pltpu
from jax.experimental.pallas import tpu_sc as plsc

N_NODES = 10000
N_EDGES = 320000
NC = 2
NS = 16
N_TILES = NC * NS
CH = 128
ROWS_PER_TILE = N_NODES // NS
ACC_ROWS = N_NODES + 8
DEG_ROWS = 640

_BLK = 1000
_GRID = N_NODES // _BLK


def _pad_chunks(v, e_per_tile, n_tiles, fill):
    n_ch = -(-e_per_tile // CH)
    pad = n_ch * CH - e_per_tile
    v = jnp.pad(v.reshape(n_tiles, e_per_tile), ((0, 0), (0, pad)),
                constant_values=fill)
    return v.reshape(n_tiles, n_ch, CH), n_ch


def _make_sc_colsplit():
    d = 32
    e_per_tile = N_EDGES // NS
    n_ch = -(-e_per_tile // CH)
    n_ch = (e_per_tile + CH - 1) // CH
    if n_ch % 2:
        n_ch += 1
    mesh = plsc.VectorSubcoreMesh(
        core_axis_name="c", subcore_axis_name="s", num_cores=NC, num_subcores=NS)

    out_type = [
        jax.ShapeDtypeStruct((NC, NS, ROWS_PER_TILE, d), jnp.float32),
        jax.ShapeDtypeStruct((NC, DEG_ROWS, 16), jnp.float32),
    ]
    scratch = [
        pltpu.VMEM((n_ch, CH), jnp.int32),
        pltpu.VMEM((n_ch, CH), jnp.int32),
        pltpu.VMEM((n_ch, CH), jnp.int32),
        pltpu.VMEM((CH, d), jnp.float32),
        pltpu.VMEM((CH, d), jnp.float32),
        pltpu.VMEM_SHARED((ACC_ROWS, d), jnp.float32),
        pltpu.VMEM_SHARED((N_NODES, d), jnp.float32),
        pltpu.VMEM((DEG_ROWS, 16), jnp.float32),
        pltpu.VMEM((DEG_ROWS // CH, CH), jnp.int32),
        pltpu.VMEM_SHARED((DEG_ROWS, 16), jnp.float32),
        pltpu.SemaphoreType.DMA,
        pltpu.SemaphoreType.DMA,
    ]

    @functools.partial(
        pl.kernel,
        out_type=out_type,
        mesh=mesh,
        scratch_types=scratch,
        compiler_params=pltpu.CompilerParams(use_tc_tiling_on_sc=False,
                                            needs_layout_passes=False),
    )
    def sc_agg(feat2, edges, agg_out, deg_out, epk, srcv, dstv, rows0, rows1,
               agg_sh, feat_sh, deg2d, rowidx, deg_sh, sem0, sem1):
        c = lax.axis_index("c")
        s = lax.axis_index("s")
        r0 = s * ROWS_PER_TILE
        z16 = jnp.zeros((16,), jnp.float32)

        pltpu.sync_copy(feat2.at[c, pl.ds(r0, ROWS_PER_TILE)],
                        feat_sh.at[pl.ds(r0, ROWS_PER_TILE)])
        pltpu.sync_copy(edges.at[s], epk)

        n16 = d // 16

        def zrow(g, carry):
            rows0[lax.div(g, n16), pl.ds(lax.rem(g, n16) * 16, 16)] = z16
            return carry

        lax.fori_loop(0, CH * n16, zrow, 0)
        off = 0
        while off < ROWS_PER_TILE:
            sz = min(CH, ROWS_PER_TILE - off)
            pltpu.sync_copy(rows0.at[pl.ds(0, sz)],
                            agg_sh.at[pl.ds(r0 + off, sz)])
            off += sz

        def zdeg(i, carry):
            deg2d[i] = z16
            return carry

        lax.fori_loop(0, DEG_ROWS, zdeg, 0)

        @pl.when((s == 0) & (c == 0))
        def _():
            pltpu.sync_copy(deg2d, deg_sh)

        def irow(g, carry):
            rowidx[lax.div(g, 8), pl.ds(lax.rem(g, 8) * 16, 16)] = (
                lax.iota(jnp.int32, 16) + g * 16)
            return carry

        lax.fori_loop(0, DEG_ROWS // 16, irow, 0)

        def unpack(g, carry):
            row = lax.div(g, 8)
            col = lax.rem(g, 8) * 16
            e = epk[row, pl.ds(col, 16)]
            srcv[row, pl.ds(col, 16)] = e & 0xFFFF
            dstv[row, pl.ds(col, 16)] = lax.shift_right_logical(e, 16)
            return carry

        lax.fori_loop(0, n_ch * CH // 16, unpack, 0)
        plsc.subcore_barrier()

        pltpu.async_copy(feat_sh.at[srcv.at[0]], rows0, sem0)

        def drain(buf, sem):
            pltpu.make_async_copy(feat2.at[0, pl.ds(0, CH)], buf, sem).wait()

        def hist(k):
            @pl.when(c == 0)
            def _():
                for g in range(CH // 16):
                    di = dstv[k, pl.ds(g * 16, 16)]
                    q = lax.shift_right_logical(di, 4)
                    r = di & 15
                    plsc.addupdate_scatter(deg2d, [q, r],
                                           jnp.ones((16,), jnp.float32))

        def body(j, carry):
            k = 2 * j
            pltpu.async_copy(feat_sh.at[srcv.at[k + 1]], rows1, sem1)
            hist(k)
            drain(rows0, sem0)
            pltpu.sync_copy(rows0, agg_sh.at[dstv.at[k]], add=True)

            @pl.when(j < n_ch // 2 - 1)
            def _():
                pltpu.async_copy(feat_sh.at[srcv.at[k + 2]], rows0, sem0)

            hist(k + 1)
            drain(rows1, sem1)
            pltpu.sync_copy(rows1, agg_sh.at[dstv.at[k + 1]], add=True)
            return carry

        lax.fori_loop(0, n_ch // 2, body, 0)

        @pl.when(c == 0)
        def _():
            for t in range(DEG_ROWS // CH):
                pltpu.sync_copy(deg2d.at[pl.ds(t * CH, CH)],
                                deg_sh.at[rowidx.at[t]], add=True)
        plsc.subcore_barrier()
        pltpu.sync_copy(agg_sh.at[pl.ds(r0, ROWS_PER_TILE)], agg_out.at[c, s])

        @pl.when((s == 0) & (c == 0))
        def _():
            pltpu.sync_copy(deg_sh, deg_out.at[0])

    return sc_agg, n_ch


_sc_l1, N_CH1 = _make_sc_colsplit()
E_PER_TILE1 = N_EDGES // NS
E_PAD1 = N_CH1 * CH

E_PER_TILE2 = N_EDGES // N_TILES
N_CH2 = 80
E_PAD2 = N_CH2 * CH


def _make_sc_edgesplit(d):
    mesh = plsc.VectorSubcoreMesh(
        core_axis_name="c", subcore_axis_name="s", num_cores=NC, num_subcores=NS)
    scratch = [
        pltpu.VMEM((N_CH2, CH), jnp.int32),
        pltpu.VMEM((N_CH2, CH), jnp.int32),
        pltpu.VMEM((N_CH2, CH), jnp.int32),
        pltpu.VMEM((CH, d), jnp.float32),
        pltpu.VMEM((CH, d), jnp.float32),
        pltpu.VMEM_SHARED((ACC_ROWS, d), jnp.float32),
        pltpu.VMEM_SHARED((N_NODES, d), jnp.float32),
        pltpu.SemaphoreType.DMA,
        pltpu.SemaphoreType.DMA,
    ]

    @functools.partial(
        pl.kernel,
        out_type=jax.ShapeDtypeStruct((NC, NS, ROWS_PER_TILE, d), jnp.float32),
        mesh=mesh,
        scratch_types=scratch,
        compiler_params=pltpu.CompilerParams(use_tc_tiling_on_sc=False,
                                            needs_layout_passes=False),
    )
    def sc_agg(feat, edges, agg_out, epk, srcv, dstv, rows0, rows1,
               agg_sh, feat_sh, sem0, sem1):
        c = lax.axis_index("c")
        s = lax.axis_index("s")
        wid = s * NC + c
        r0 = s * ROWS_PER_TILE
        z16 = jnp.zeros((16,), jnp.float32)

        pltpu.sync_copy(feat.at[pl.ds(r0, ROWS_PER_TILE)],
                        feat_sh.at[pl.ds(r0, ROWS_PER_TILE)])
        pltpu.sync_copy(edges.at[wid], epk)

        n16 = d // 16

        def zrow(g, carry):
            rows0[lax.div(g, n16), pl.ds(lax.rem(g, n16) * 16, 16)] = z16
            return carry

        lax.fori_loop(0, CH * n16, zrow, 0)
        off = 0
        while off < ROWS_PER_TILE:
            sz = min(CH, ROWS_PER_TILE - off)
            pltpu.sync_copy(rows0.at[pl.ds(0, sz)],
                            agg_sh.at[pl.ds(r0 + off, sz)])
            off += sz

        def unpack(g, carry):
            row = lax.div(g, 8)
            col = lax.rem(g, 8) * 16
            e = epk[row, pl.ds(col, 16)]
            srcv[row, pl.ds(col, 16)] = e & 0xFFFF
            dstv[row, pl.ds(col, 16)] = lax.shift_right_logical(e, 16)
            return carry

        lax.fori_loop(0, N_CH2 * CH // 16, unpack, 0)
        plsc.subcore_barrier()

        pltpu.async_copy(feat_sh.at[srcv.at[0]], rows0, sem0)

        def drain(buf, sem):
            pltpu.make_async_copy(feat.at[pl.ds(0, CH)], buf, sem).wait()

        def body(j, carry):
            k = 2 * j
            pltpu.async_copy(feat_sh.at[srcv.at[k + 1]], rows1, sem1)
            drain(rows0, sem0)
            pltpu.sync_copy(rows0, agg_sh.at[dstv.at[k]], add=True)

            @pl.when(j < N_CH2 // 2 - 1)
            def _():
                pltpu.async_copy(feat_sh.at[srcv.at[k + 2]], rows0, sem0)

            drain(rows1, sem1)
            pltpu.sync_copy(rows1, agg_sh.at[dstv.at[k + 1]], add=True)
            return carry

        lax.fori_loop(0, N_CH2 // 2, body, 0)
        plsc.subcore_barrier()
        pltpu.sync_copy(agg_sh.at[pl.ds(r0, ROWS_PER_TILE)], agg_out.at[c, s])

    return sc_agg


_sc_l2 = _make_sc_edgesplit(32)


def _tc1_body(x_ref, wla_ref, wlb_ref, wr_ref, ps_ref, r_ref):
    xb = x_ref[...]
    ps_ref[0] = jnp.dot(xb, wla_ref[...], preferred_element_type=jnp.float32)
    ps_ref[1] = jnp.dot(xb, wlb_ref[...], preferred_element_type=jnp.float32)
    r_ref[...] = jnp.dot(xb, wr_ref[...], preferred_element_type=jnp.float32)


def _tc1(x, wla, wlb, wr1):
    return pl.pallas_call(
        _tc1_body,
        grid=(_GRID,),
        in_specs=[
            pl.BlockSpec((_BLK, 128), lambda i: (i, 0)),
            pl.BlockSpec((128, 32), lambda i: (0, 0)),
            pl.BlockSpec((128, 32), lambda i: (0, 0)),
            pl.BlockSpec((128, 64), lambda i: (0, 0)),
        ],
        out_specs=[
            pl.BlockSpec((NC, _BLK, 32), lambda i: (0, i, 0)),
            pl.BlockSpec((_BLK, 64), lambda i: (i, 0)),
        ],
        out_shape=[
            jax.ShapeDtypeStruct((NC, N_NODES, 32), jnp.float32),
            jax.ShapeDtypeStruct((N_NODES, 64), jnp.float32),
        ],
    )(x, wla, wlb, wr1)


def _tc2_body(agg_ref, deg_ref, r1_ref, b1_ref, wl2a_ref, wl2b_ref,
              wr2a_ref, wr2b_ref, p2_ref, r2_ref, inv8_ref):
    deg = deg_ref[...]
    inv = 1.0 / jnp.maximum(deg, 1.0)
    r1b = r1_ref[...]
    b1b = b1_ref[...]
    h0 = jnp.maximum(agg_ref[0] * inv + r1b[:, :32] + b1b[:, :32], 0.0)
    h1 = jnp.maximum(agg_ref[1] * inv + r1b[:, 32:] + b1b[:, 32:], 0.0)
    p2_ref[...] = (jnp.dot(h0, wl2a_ref[...], preferred_element_type=jnp.float32)
                   + jnp.dot(h1, wl2b_ref[...], preferred_element_type=jnp.float32))
    r2_ref[...] = (jnp.dot(h0, wr2a_ref[...], preferred_element_type=jnp.float32)
                   + jnp.dot(h1, wr2b_ref[...], preferred_element_type=jnp.float32))
    inv8_ref[...] = jnp.broadcast_to(inv, (inv.shape[0], 8))


def _tc2(agg1, deg1, r1, b1r, wl2a, wl2b, wr2a, wr2b):
    return pl.pallas_call(
        _tc2_body,
        grid=(_GRID,),
        in_specs=[
            pl.BlockSpec((NC, _BLK, 32), lambda i: (0, i, 0)),
            pl.BlockSpec((_BLK, 1), lambda i: (i, 0)),
            pl.BlockSpec((_BLK, 64), lambda i: (i, 0)),
            pl.BlockSpec((1, 64), lambda i: (0, 0)),
            pl.BlockSpec((32, 32), lambda i: (0, 0)),
            pl.BlockSpec((32, 32), lambda i: (0, 0)),
            pl.BlockSpec((32, 32), lambda i: (0, 0)),
            pl.BlockSpec((32, 32), lambda i: (0, 0)),
        ],
        out_specs=[
            pl.BlockSpec((_BLK, 32), lambda i: (i, 0)),
            pl.BlockSpec((_BLK, 32), lambda i: (i, 0)),
            pl.BlockSpec((_BLK, 8), lambda i: (i, 0)),
        ],
        out_shape=[
            jax.ShapeDtypeStruct((N_NODES, 32), jnp.float32),
            jax.ShapeDtypeStruct((N_NODES, 32), jnp.float32),
            jax.ShapeDtypeStruct((N_NODES, 8), jnp.float32),
        ],
    )(agg1, deg1, r1, b1r, wl2a, wl2b, wr2a, wr2b)


def _tc3_body(agg_ref, r2_ref, inv8_ref, b2_ref, w3_ref, b3_ref, o_ref):
    inv = inv8_ref[...][:, :1]
    h2 = jnp.maximum((agg_ref[0] + agg_ref[1]) * inv + r2_ref[...] + b2_ref[...], 0.0)
    o_ref[...] = jnp.dot(h2, w3_ref[...], preferred_element_type=jnp.float32) + b3_ref[...]


def _tc3(agg2, r2, inv8, b2r, w3pad, b3r):
    return pl.pallas_call(
        _tc3_body,
        grid=(_GRID,),
        in_specs=[
            pl.BlockSpec((NC, _BLK, 32), lambda i: (0, i, 0)),
            pl.BlockSpec((_BLK, 32), lambda i: (i, 0)),
            pl.BlockSpec((_BLK, 8), lambda i: (i, 0)),
            pl.BlockSpec((1, 32), lambda i: (0, 0)),
            pl.BlockSpec((32, 128), lambda i: (0, 0)),
            pl.BlockSpec((1, 128), lambda i: (0, 0)),
        ],
        out_specs=pl.BlockSpec((_BLK, 128), lambda i: (i, 0)),
        out_shape=jax.ShapeDtypeStruct((N_NODES, 128), jnp.float32),
    )(agg2, r2, inv8, b2r, w3pad, b3r)


def kernel(x, edge_index, W_l1, W_r1, b1, W_l2, W_r2, b2, W3, b3):
    src = edge_index[0].astype(jnp.int32)
    dst = edge_index[1].astype(jnp.int32)
    packed = src | (dst << 16)

    pad1 = E_PAD1 - E_PER_TILE1
    e1 = jnp.pad(packed.reshape(NS, E_PER_TILE1), ((0, 0), (0, pad1)),
                 constant_values=N_NODES << 16).reshape(NS, N_CH1, CH)
    pad2 = E_PAD2 - E_PER_TILE2
    e2 = jnp.pad(packed.reshape(N_TILES, E_PER_TILE2), ((0, 0), (0, pad2)),
                 constant_values=N_NODES << 16).reshape(N_TILES, N_CH2, CH)

    p1s, r1 = _tc1(x, W_l1[:, :32], W_l1[:, 32:], W_r1)
    agg1, deg_raw = _sc_l1(p1s, e1)
    agg1 = agg1.reshape(NC, N_NODES, 32)
    deg1 = deg_raw[0].reshape(DEG_ROWS * 16, 1)[:N_NODES]
    p2, r2, inv8 = _tc2(agg1, deg1, r1, b1.reshape(1, 64),
                        W_l2[:32], W_l2[32:], W_r2[:32], W_r2[32:])
    agg2 = _sc_l2(p2, e2).reshape(NC, N_NODES, 32)
    o = _tc3(agg2, r2, inv8, b2.reshape(1, 32),
             jnp.pad(W3, ((0, 0), (0, 127))),
             jnp.broadcast_to(b3.reshape(1, 1), (1, 128)))
    return o[:, :1]


# speedup vs baseline: 14.3242x; 1.0199x over previous
import functools

import jax
import jax.numpy as jnp
from jax import lax
from jax.experimental import pallas as pl
from jax.experimental.pallas import tpu as pltpu
from jax.experimental.pallas import tpu_sc as plsc

N_NODES = 10000
N_EDGES = 320000
NC = 2
NS = 16
N_TILES = NC * NS
CH = 128
ROWS_PER_TILE = N_NODES // NS
ACC_ROWS = N_NODES + 8
DEG_ROWS = 640

_BLK = 1000
_GRID = N_NODES // _BLK


def _pad_chunks(v, e_per_tile, n_tiles, fill):
    n_ch = -(-e_per_tile // CH)
    pad = n_ch * CH - e_per_tile
    v = jnp.pad(v.reshape(n_tiles, e_per_tile), ((0, 0), (0, pad)),
                constant_values=fill)
    return v.reshape(n_tiles, n_ch, CH), n_ch


def _make_sc_colsplit():
    d = 32
    e_per_tile = N_EDGES // NS
    n_ch = -(-e_per_tile // CH)
    n_ch = (e_per_tile + CH - 1) // CH
    if n_ch % 2:
        n_ch += 1
    mesh = plsc.VectorSubcoreMesh(
        core_axis_name="c", subcore_axis_name="s", num_cores=NC, num_subcores=NS)

    out_type = [
        jax.ShapeDtypeStruct((NC, NS, ROWS_PER_TILE, d), jnp.float32),
        jax.ShapeDtypeStruct((NC, DEG_ROWS, 16), jnp.float32),
    ]
    scratch = [
        pltpu.VMEM((n_ch, CH), jnp.int32),
        pltpu.VMEM((n_ch, CH), jnp.int32),
        pltpu.VMEM((n_ch, CH), jnp.int32),
        pltpu.VMEM((CH, d), jnp.float32),
        pltpu.VMEM((CH, d), jnp.float32),
        pltpu.VMEM_SHARED((ACC_ROWS, d), jnp.float32),
        pltpu.VMEM_SHARED((N_NODES, d), jnp.float32),
        pltpu.VMEM((DEG_ROWS, 16), jnp.float32),
        pltpu.VMEM((DEG_ROWS // CH, CH), jnp.int32),
        pltpu.VMEM_SHARED((DEG_ROWS, 16), jnp.float32),
        pltpu.SemaphoreType.DMA,
        pltpu.SemaphoreType.DMA,
    ]

    @functools.partial(
        pl.kernel,
        out_type=out_type,
        mesh=mesh,
        scratch_types=scratch,
        compiler_params=pltpu.CompilerParams(use_tc_tiling_on_sc=False,
                                            needs_layout_passes=False),
    )
    def sc_agg(feat2, edges, agg_out, deg_out, epk, srcv, dstv, rows0, rows1,
               agg_sh, feat_sh, deg2d, rowidx, deg_sh, sem0, sem1):
        c = lax.axis_index("c")
        s = lax.axis_index("s")
        r0 = s * ROWS_PER_TILE
        z16 = jnp.zeros((16,), jnp.float32)

        pltpu.sync_copy(feat2.at[c, pl.ds(r0, ROWS_PER_TILE)],
                        feat_sh.at[pl.ds(r0, ROWS_PER_TILE)])
        pltpu.sync_copy(edges.at[s], epk)

        n16 = d // 16

        def zrow(g, carry):
            rows0[lax.div(g, n16), pl.ds(lax.rem(g, n16) * 16, 16)] = z16
            return carry

        lax.fori_loop(0, CH * n16, zrow, 0)
        off = 0
        while off < ROWS_PER_TILE:
            sz = min(CH, ROWS_PER_TILE - off)
            pltpu.sync_copy(rows0.at[pl.ds(0, sz)],
                            agg_sh.at[pl.ds(r0 + off, sz)])
            off += sz

        def zdeg(i, carry):
            deg2d[i] = z16
            return carry

        lax.fori_loop(0, DEG_ROWS, zdeg, 0)

        @pl.when((s == 0) & (c == 0))
        def _():
            pltpu.sync_copy(deg2d, deg_sh)

        def irow(g, carry):
            rowidx[lax.div(g, 8), pl.ds(lax.rem(g, 8) * 16, 16)] = (
                lax.iota(jnp.int32, 16) + g * 16)
            return carry

        lax.fori_loop(0, DEG_ROWS // 16, irow, 0)

        def unpack_row(row):
            for g in range(8):
                e = epk[row, pl.ds(g * 16, 16)]
                srcv[row, pl.ds(g * 16, 16)] = e & 0xFFFF
                dstv[row, pl.ds(g * 16, 16)] = lax.shift_right_logical(e, 16)

        for row in range(4):
            unpack_row(row)
        plsc.subcore_barrier()

        pltpu.async_copy(feat_sh.at[srcv.at[0]], rows0, sem0)

        def drain(buf, sem):
            pltpu.make_async_copy(feat2.at[0, pl.ds(0, CH)], buf, sem).wait()

        def hist(k):
            @pl.when(c == 0)
            def _():
                for g in range(CH // 16):
                    di = dstv[k, pl.ds(g * 16, 16)]
                    q = lax.shift_right_logical(di, 4)
                    r = di & 15
                    plsc.addupdate_scatter(deg2d, [q, r],
                                           jnp.ones((16,), jnp.float32))

        def body(j, carry):
            k = 2 * j
            pltpu.async_copy(feat_sh.at[srcv.at[k + 1]], rows1, sem1)
            unpack_row(jnp.minimum(k + 3, n_ch - 1))
            hist(k)
            drain(rows0, sem0)
            pltpu.sync_copy(rows0, agg_sh.at[dstv.at[k]], add=True)

            @pl.when(j < n_ch // 2 - 1)
            def _():
                pltpu.async_copy(feat_sh.at[srcv.at[k + 2]], rows0, sem0)

            unpack_row(jnp.minimum(k + 4, n_ch - 1))
            hist(k + 1)
            drain(rows1, sem1)
            pltpu.sync_copy(rows1, agg_sh.at[dstv.at[k + 1]], add=True)
            return carry

        lax.fori_loop(0, n_ch // 2, body, 0)

        @pl.when(c == 0)
        def _():
            for t in range(DEG_ROWS // CH):
                pltpu.sync_copy(deg2d.at[pl.ds(t * CH, CH)],
                                deg_sh.at[rowidx.at[t]], add=True)
        plsc.subcore_barrier()
        pltpu.sync_copy(agg_sh.at[pl.ds(r0, ROWS_PER_TILE)], agg_out.at[c, s])

        @pl.when((s == 0) & (c == 0))
        def _():
            pltpu.sync_copy(deg_sh, deg_out.at[0])

    return sc_agg, n_ch


_sc_l1, N_CH1 = _make_sc_colsplit()
E_PER_TILE1 = N_EDGES // NS
E_PAD1 = N_CH1 * CH

E_PER_TILE2 = N_EDGES // N_TILES
N_CH2 = 80
E_PAD2 = N_CH2 * CH


def _make_sc_edgesplit(d):
    mesh = plsc.VectorSubcoreMesh(
        core_axis_name="c", subcore_axis_name="s", num_cores=NC, num_subcores=NS)
    scratch = [
        pltpu.VMEM((N_CH2, CH), jnp.int32),
        pltpu.VMEM((N_CH2, CH), jnp.int32),
        pltpu.VMEM((N_CH2, CH), jnp.int32),
        pltpu.VMEM((CH, d), jnp.float32),
        pltpu.VMEM((CH, d), jnp.float32),
        pltpu.VMEM_SHARED((ACC_ROWS, d), jnp.float32),
        pltpu.VMEM_SHARED((N_NODES, d), jnp.float32),
        pltpu.SemaphoreType.DMA,
        pltpu.SemaphoreType.DMA,
    ]

    @functools.partial(
        pl.kernel,
        out_type=jax.ShapeDtypeStruct((NC, NS, ROWS_PER_TILE, d), jnp.float32),
        mesh=mesh,
        scratch_types=scratch,
        compiler_params=pltpu.CompilerParams(use_tc_tiling_on_sc=False,
                                            needs_layout_passes=False),
    )
    def sc_agg(feat, edges, agg_out, epk, srcv, dstv, rows0, rows1,
               agg_sh, feat_sh, sem0, sem1):
        c = lax.axis_index("c")
        s = lax.axis_index("s")
        wid = s * NC + c
        r0 = s * ROWS_PER_TILE
        z16 = jnp.zeros((16,), jnp.float32)

        pltpu.sync_copy(feat.at[pl.ds(r0, ROWS_PER_TILE)],
                        feat_sh.at[pl.ds(r0, ROWS_PER_TILE)])
        pltpu.sync_copy(edges.at[wid], epk)

        n16 = d // 16

        def zrow(g, carry):
            rows0[lax.div(g, n16), pl.ds(lax.rem(g, n16) * 16, 16)] = z16
            return carry

        lax.fori_loop(0, CH * n16, zrow, 0)
        off = 0
        while off < ROWS_PER_TILE:
            sz = min(CH, ROWS_PER_TILE - off)
            pltpu.sync_copy(rows0.at[pl.ds(0, sz)],
                            agg_sh.at[pl.ds(r0 + off, sz)])
            off += sz

        def unpack_row(row):
            for g in range(8):
                e = epk[row, pl.ds(g * 16, 16)]
                srcv[row, pl.ds(g * 16, 16)] = e & 0xFFFF
                dstv[row, pl.ds(g * 16, 16)] = lax.shift_right_logical(e, 16)

        for row in range(4):
            unpack_row(row)
        plsc.subcore_barrier()

        pltpu.async_copy(feat_sh.at[srcv.at[0]], rows0, sem0)

        def drain(buf, sem):
            pltpu.make_async_copy(feat.at[pl.ds(0, CH)], buf, sem).wait()

        def body(j, carry):
            k = 2 * j
            pltpu.async_copy(feat_sh.at[srcv.at[k + 1]], rows1, sem1)
            unpack_row(jnp.minimum(k + 3, N_CH2 - 1))
            drain(rows0, sem0)
            pltpu.sync_copy(rows0, agg_sh.at[dstv.at[k]], add=True)

            @pl.when(j < N_CH2 // 2 - 1)
            def _():
                pltpu.async_copy(feat_sh.at[srcv.at[k + 2]], rows0, sem0)

            unpack_row(jnp.minimum(k + 4, N_CH2 - 1))
            drain(rows1, sem1)
            pltpu.sync_copy(rows1, agg_sh.at[dstv.at[k + 1]], add=True)
            return carry

        lax.fori_loop(0, N_CH2 // 2, body, 0)
        plsc.subcore_barrier()
        pltpu.sync_copy(agg_sh.at[pl.ds(r0, ROWS_PER_TILE)], agg_out.at[c, s])

    return sc_agg


_sc_l2 = _make_sc_edgesplit(32)


def _tc1_body(x_ref, wla_ref, wlb_ref, wr_ref, ps_ref, r_ref):
    xb = x_ref[...]
    ps_ref[0] = jnp.dot(xb, wla_ref[...], preferred_element_type=jnp.float32)
    ps_ref[1] = jnp.dot(xb, wlb_ref[...], preferred_element_type=jnp.float32)
    r_ref[...] = jnp.dot(xb, wr_ref[...], preferred_element_type=jnp.float32)


def _tc1(x, wla, wlb, wr1):
    return pl.pallas_call(
        _tc1_body,
        grid=(_GRID,),
        in_specs=[
            pl.BlockSpec((_BLK, 128), lambda i: (i, 0)),
            pl.BlockSpec((128, 32), lambda i: (0, 0)),
            pl.BlockSpec((128, 32), lambda i: (0, 0)),
            pl.BlockSpec((128, 64), lambda i: (0, 0)),
        ],
        out_specs=[
            pl.BlockSpec((NC, _BLK, 32), lambda i: (0, i, 0)),
            pl.BlockSpec((_BLK, 64), lambda i: (i, 0)),
        ],
        out_shape=[
            jax.ShapeDtypeStruct((NC, N_NODES, 32), jnp.float32),
            jax.ShapeDtypeStruct((N_NODES, 64), jnp.float32),
        ],
    )(x, wla, wlb, wr1)


def _tc2_body(agg_ref, deg_ref, r1_ref, b1_ref, wl2a_ref, wl2b_ref,
              wr2a_ref, wr2b_ref, p2_ref, r2_ref, inv8_ref):
    deg = deg_ref[...]
    inv = 1.0 / jnp.maximum(deg, 1.0)
    r1b = r1_ref[...]
    b1b = b1_ref[...]
    h0 = jnp.maximum(agg_ref[0] * inv + r1b[:, :32] + b1b[:, :32], 0.0)
    h1 = jnp.maximum(agg_ref[1] * inv + r1b[:, 32:] + b1b[:, 32:], 0.0)
    p2_ref[...] = (jnp.dot(h0, wl2a_ref[...], preferred_element_type=jnp.float32)
                   + jnp.dot(h1, wl2b_ref[...], preferred_element_type=jnp.float32))
    r2_ref[...] = (jnp.dot(h0, wr2a_ref[...], preferred_element_type=jnp.float32)
                   + jnp.dot(h1, wr2b_ref[...], preferred_element_type=jnp.float32))
    inv8_ref[...] = jnp.broadcast_to(inv, (inv.shape[0], 8))


def _tc2(agg1, deg1, r1, b1r, wl2a, wl2b, wr2a, wr2b):
    return pl.pallas_call(
        _tc2_body,
        grid=(_GRID,),
        in_specs=[
            pl.BlockSpec((NC, _BLK, 32), lambda i: (0, i, 0)),
            pl.BlockSpec((_BLK, 1), lambda i: (i, 0)),
            pl.BlockSpec((_BLK, 64), lambda i: (i, 0)),
            pl.BlockSpec((1, 64), lambda i: (0, 0)),
            pl.BlockSpec((32, 32), lambda i: (0, 0)),
            pl.BlockSpec((32, 32), lambda i: (0, 0)),
            pl.BlockSpec((32, 32), lambda i: (0, 0)),
            pl.BlockSpec((32, 32), lambda i: (0, 0)),
        ],
        out_specs=[
            pl.BlockSpec((_BLK, 32), lambda i: (i, 0)),
            pl.BlockSpec((_BLK, 32), lambda i: (i, 0)),
            pl.BlockSpec((_BLK, 8), lambda i: (i, 0)),
        ],
        out_shape=[
            jax.ShapeDtypeStruct((N_NODES, 32), jnp.float32),
            jax.ShapeDtypeStruct((N_NODES, 32), jnp.float32),
            jax.ShapeDtypeStruct((N_NODES, 8), jnp.float32),
        ],
    )(agg1, deg1, r1, b1r, wl2a, wl2b, wr2a, wr2b)


def _tc3_body(agg_ref, r2_ref, inv8_ref, b2_ref, w3_ref, b3_ref, o_ref):
    inv = inv8_ref[...][:, :1]
    h2 = jnp.maximum((agg_ref[0] + agg_ref[1]) * inv + r2_ref[...] + b2_ref[...], 0.0)
    o_ref[...] = jnp.dot(h2, w3_ref[...], preferred_element_type=jnp.float32) + b3_ref[...]


def _tc3(agg2, r2, inv8, b2r, w3pad, b3r):
    return pl.pallas_call(
        _tc3_body,
        grid=(_GRID,),
        in_specs=[
            pl.BlockSpec((NC, _BLK, 32), lambda i: (0, i, 0)),
            pl.BlockSpec((_BLK, 32), lambda i: (i, 0)),
            pl.BlockSpec((_BLK, 8), lambda i: (i, 0)),
            pl.BlockSpec((1, 32), lambda i: (0, 0)),
            pl.BlockSpec((32, 8), lambda i: (0, 0)),
            pl.BlockSpec((1, 8), lambda i: (0, 0)),
        ],
        out_specs=pl.BlockSpec((_BLK, 8), lambda i: (i, 0)),
        out_shape=jax.ShapeDtypeStruct((N_NODES, 8), jnp.float32),
    )(agg2, r2, inv8, b2r, w3pad, b3r)


def kernel(x, edge_index, W_l1, W_r1, b1, W_l2, W_r2, b2, W3, b3):
    src = edge_index[0].astype(jnp.int32)
    dst = edge_index[1].astype(jnp.int32)
    packed = src | (dst << 16)

    pad1 = E_PAD1 - E_PER_TILE1
    e1 = jnp.pad(packed.reshape(NS, E_PER_TILE1), ((0, 0), (0, pad1)),
                 constant_values=N_NODES << 16).reshape(NS, N_CH1, CH)
    pad2 = E_PAD2 - E_PER_TILE2
    e2 = jnp.pad(packed.reshape(N_TILES, E_PER_TILE2), ((0, 0), (0, pad2)),
                 constant_values=N_NODES << 16).reshape(N_TILES, N_CH2, CH)

    p1s, r1 = _tc1(x, W_l1[:, :32], W_l1[:, 32:], W_r1)
    agg1, deg_raw = _sc_l1(p1s, e1)
    agg1 = agg1.reshape(NC, N_NODES, 32)
    deg1 = deg_raw[0].reshape(DEG_ROWS * 16, 1)[:N_NODES]
    p2, r2, inv8 = _tc2(agg1, deg1, r1, b1.reshape(1, 64),
                        W_l2[:32], W_l2[32:], W_r2[:32], W_r2[32:])
    agg2 = _sc_l2(p2, e2).reshape(NC, N_NODES, 32)
    o = _tc3(agg2, r2, inv8, b2.reshape(1, 32),
             jnp.pad(W3, ((0, 0), (0, 7))),
             jnp.broadcast_to(b3.reshape(1, 1), (1, 8)))
    return o[:, :1]
